# baseline (speedup 1.0000x reference)
import functools
import math

import jax
import jax.numpy as jnp
from jax import lax
from jax.experimental import pallas as pl
from jax.experimental.pallas import tpu as pltpu
from jax.experimental.pallas import tpu_sc as plsc

_SIZE = 32000
_PAD = 0
_SMOOTH = 0.1
_CONF = 1.0 - _SMOOTH
_EPS = _SMOOTH / (_SIZE - 2)
_N = 4096
_C1 = _EPS * math.log(_EPS) * (_SIZE - 2) + _CONF * math.log(_CONF)

_NW = 32
_R_SC = 1024
_R_TC = _N - _R_SC
_RPW = _R_SC // _NW
_PICKS_PW = _N // _NW
_BM = 128
_BN = _SIZE


def _tc_kernel(t_ref, x_ref, o_ref):
    i = pl.program_id(0)

    @pl.when(i == 0)
    def _():
        o_ref[...] = jnp.zeros_like(o_ref)

    t = t_ref[...]
    rowsum = jnp.sum(x_ref[...], axis=1, keepdims=True)
    acc = jnp.sum(jnp.where(t != _PAD, -_EPS, 0.0) * rowsum)
    o_ref[...] += acc.reshape(1, 1)


def _tc_call(t32, x):
    return pl.pallas_call(
        _tc_kernel,
        grid=(_R_TC // _BM,),
        in_specs=[
            pl.BlockSpec((_BM, 1), lambda i: (i, 0)),
            pl.BlockSpec((_BM, _BN), lambda i: (i, 0)),
        ],
        out_specs=pl.BlockSpec((1, 1), lambda i: (0, 0)),
        out_shape=jax.ShapeDtypeStruct((1, 1), jnp.float32),
    )(t32.reshape(_N, 1), x)


def _sc_body(x_hbm, xflat_hbm, t_hbm, out_hbm,
             rowbuf, rowbuf2, idxt, idx0, picks, col0s, tpick, trows, outv,
             sem, sem0, sem1):
    wid = lax.axis_index("c") * 16 + lax.axis_index("s")
    acc = jnp.zeros((16,), jnp.float32)

    pick_base = wid * _PICKS_PW
    pltpu.sync_copy(t_hbm.at[pl.ds(pick_base, _PICKS_PW)], tpick)
    for k in range(_PICKS_PW // 16):
        rowv = (pick_base + k * 16 + lax.iota(jnp.int32, 16)) * _SIZE
        idx0[pl.ds(k * 16, 16)] = rowv
        idxt[pl.ds(k * 16, 16)] = rowv + tpick[pl.ds(k * 16, 16)]
    pltpu.async_copy(xflat_hbm.at[idxt], picks, sem).wait()
    pltpu.async_copy(xflat_hbm.at[idx0], col0s, sem).wait()
    for k in range(_PICKS_PW // 16):
        tk = tpick[pl.ds(k * 16, 16)]
        corr = ((_EPS - _CONF) * picks[pl.ds(k * 16, 16)]
                + _EPS * col0s[pl.ds(k * 16, 16)] + _C1)
        acc = acc + jnp.where(tk != _PAD, corr, 0.0)

    row_base = _R_TC + wid * _RPW
    pltpu.sync_copy(t_hbm.at[pl.ds(row_base, _RPW)], trows)

    def _row_sum(buf):
        def chunk_body(k, accs):
            a0, a1, a2, a3 = accs
            b = k * 256
            for u in range(4):
                a0 = a0 + buf[pl.ds(b + u * 64, 16)]
                a1 = a1 + buf[pl.ds(b + u * 64 + 16, 16)]
                a2 = a2 + buf[pl.ds(b + u * 64 + 32, 16)]
                a3 = a3 + buf[pl.ds(b + u * 64 + 48, 16)]
            return (a0, a1, a2, a3)

        z = jnp.zeros((16,), jnp.float32)
        a0, a1, a2, a3 = lax.fori_loop(0, _SIZE // 256, chunk_body,
                                       (z, z, z, z))
        return (a0 + a1) + (a2 + a3)

    bufs = (rowbuf, rowbuf2)
    sems = (sem0, sem1)
    pltpu.async_copy(x_hbm.at[row_base], rowbuf, sem0)

    def group_body(g, acc):
        tv = trows[pl.ds(g * 16, 16)]
        wv = jnp.where(tv != _PAD, jnp.float32(-_EPS), jnp.float32(0.0))
        r0 = row_base + g * 16
        for rr in range(16):
            cur, nxt = bufs[rr % 2], bufs[(rr + 1) % 2]
            scur, snxt = sems[rr % 2], sems[(rr + 1) % 2]
            nxt_row = jnp.minimum(r0 + rr + 1, _N - 1)
            pltpu.async_copy(x_hbm.at[nxt_row], nxt, snxt)
            pltpu.make_async_copy(x_hbm.at[0], cur, scur).wait()
            acc = acc + wv[rr] * _row_sum(cur)
        return acc

    acc = lax.fori_loop(0, _RPW // 16, group_body, acc)
    pltpu.make_async_copy(x_hbm.at[0], rowbuf, sem0).wait()

    outv[...] = acc
    pltpu.sync_copy(outv, out_hbm.at[wid])


def _sc_call(x, xflat, t32):
    mesh = plsc.VectorSubcoreMesh(core_axis_name="c", subcore_axis_name="s")
    f = pl.kernel(
        _sc_body,
        mesh=mesh,
        out_type=jax.ShapeDtypeStruct((_NW, 16), jnp.float32),
        scratch_types=[
            pltpu.VMEM((_SIZE,), jnp.float32),
            pltpu.VMEM((_SIZE,), jnp.float32),
            pltpu.VMEM((_PICKS_PW,), jnp.int32),
            pltpu.VMEM((_PICKS_PW,), jnp.int32),
            pltpu.VMEM((_PICKS_PW,), jnp.float32),
            pltpu.VMEM((_PICKS_PW,), jnp.float32),
            pltpu.VMEM((_PICKS_PW,), jnp.int32),
            pltpu.VMEM((_RPW,), jnp.int32),
            pltpu.VMEM((16,), jnp.float32),
            pltpu.SemaphoreType.DMA,
            pltpu.SemaphoreType.DMA,
            pltpu.SemaphoreType.DMA,
        ],
    )
    return f(x, xflat, t32)


@jax.jit
def kernel(x, target):
    t32 = target.astype(jnp.int32)
    sc_part = _sc_call(x, x.reshape(-1), t32)
    tc_part = _tc_call(t32, x)
    return tc_part[0, 0] + jnp.sum(sc_part)


# speedup vs baseline: 1.0050x; 1.0050x over previous
import functools
import math

import jax
import jax.numpy as jnp
from jax import lax
from jax.experimental import pallas as pl
from jax.experimental.pallas import tpu as pltpu
from jax.experimental.pallas import tpu_sc as plsc

_SIZE = 32000
_PAD = 0
_SMOOTH = 0.1
_CONF = 1.0 - _SMOOTH
_EPS = _SMOOTH / (_SIZE - 2)
_N = 4096
_C1 = _EPS * math.log(_EPS) * (_SIZE - 2) + _CONF * math.log(_CONF)

_NW = 32
_R_SC = 512
_R_TC = _N - _R_SC
_RPW = _R_SC // _NW
_PICKS_PW = _N // _NW
_BM = 128
_BN = _SIZE


def _tc_kernel(t_ref, x_ref, o_ref):
    i = pl.program_id(0)

    @pl.when(i == 0)
    def _():
        o_ref[...] = jnp.zeros_like(o_ref)

    t = t_ref[...]
    rowsum = jnp.sum(x_ref[...], axis=1, keepdims=True)
    acc = jnp.sum(jnp.where(t != _PAD, -_EPS, 0.0) * rowsum)
    o_ref[...] += acc.reshape(1, 1)


def _tc_call(t32, x):
    return pl.pallas_call(
        _tc_kernel,
        grid=(_R_TC // _BM,),
        in_specs=[
            pl.BlockSpec((_BM, 1), lambda i: (i, 0)),
            pl.BlockSpec((_BM, _BN), lambda i: (i, 0)),
        ],
        out_specs=pl.BlockSpec((1, 1), lambda i: (0, 0)),
        out_shape=jax.ShapeDtypeStruct((1, 1), jnp.float32),
    )(t32.reshape(_N, 1), x)


def _sc_body(x_hbm, xflat_hbm, t_hbm, out_hbm,
             rowbuf, rowbuf2, idxt, idx0, picks, col0s, tpick, trows, outv,
             sem, sem0, sem1):
    wid = lax.axis_index("c") * 16 + lax.axis_index("s")
    acc = jnp.zeros((16,), jnp.float32)

    pick_base = wid * _PICKS_PW
    pltpu.sync_copy(t_hbm.at[pl.ds(pick_base, _PICKS_PW)], tpick)
    for k in range(_PICKS_PW // 16):
        rowv = (pick_base + k * 16 + lax.iota(jnp.int32, 16)) * _SIZE
        idx0[pl.ds(k * 16, 16)] = rowv
        idxt[pl.ds(k * 16, 16)] = rowv + tpick[pl.ds(k * 16, 16)]
    pltpu.async_copy(xflat_hbm.at[idxt], picks, sem).wait()
    pltpu.async_copy(xflat_hbm.at[idx0], col0s, sem).wait()
    for k in range(_PICKS_PW // 16):
        tk = tpick[pl.ds(k * 16, 16)]
        corr = ((_EPS - _CONF) * picks[pl.ds(k * 16, 16)]
                + _EPS * col0s[pl.ds(k * 16, 16)] + _C1)
        acc = acc + jnp.where(tk != _PAD, corr, 0.0)

    row_base = _R_TC + wid * _RPW
    pltpu.sync_copy(t_hbm.at[pl.ds(row_base, _RPW)], trows)

    def _row_sum(buf):
        def chunk_body(k, accs):
            a0, a1, a2, a3 = accs
            b = k * 256
            for u in range(4):
                a0 = a0 + buf[pl.ds(b + u * 64, 16)]
                a1 = a1 + buf[pl.ds(b + u * 64 + 16, 16)]
                a2 = a2 + buf[pl.ds(b + u * 64 + 32, 16)]
                a3 = a3 + buf[pl.ds(b + u * 64 + 48, 16)]
            return (a0, a1, a2, a3)

        z = jnp.zeros((16,), jnp.float32)
        a0, a1, a2, a3 = lax.fori_loop(0, _SIZE // 256, chunk_body,
                                       (z, z, z, z))
        return (a0 + a1) + (a2 + a3)

    bufs = (rowbuf, rowbuf2)
    sems = (sem0, sem1)
    pltpu.async_copy(x_hbm.at[row_base], rowbuf, sem0)

    def group_body(g, acc):
        tv = trows[pl.ds(g * 16, 16)]
        wv = jnp.where(tv != _PAD, jnp.float32(-_EPS), jnp.float32(0.0))
        r0 = row_base + g * 16
        for rr in range(16):
            cur, nxt = bufs[rr % 2], bufs[(rr + 1) % 2]
            scur, snxt = sems[rr % 2], sems[(rr + 1) % 2]
            nxt_row = jnp.minimum(r0 + rr + 1, _N - 1)
            pltpu.async_copy(x_hbm.at[nxt_row], nxt, snxt)
            pltpu.make_async_copy(x_hbm.at[0], cur, scur).wait()
            acc = acc + wv[rr] * _row_sum(cur)
        return acc

    acc = lax.fori_loop(0, _RPW // 16, group_body, acc)
    pltpu.make_async_copy(x_hbm.at[0], rowbuf, sem0).wait()

    outv[...] = acc
    pltpu.sync_copy(outv, out_hbm.at[wid])


def _sc_call(x, xflat, t32):
    mesh = plsc.VectorSubcoreMesh(core_axis_name="c", subcore_axis_name="s")
    f = pl.kernel(
        _sc_body,
        mesh=mesh,
        out_type=jax.ShapeDtypeStruct((_NW, 16), jnp.float32),
        scratch_types=[
            pltpu.VMEM((_SIZE,), jnp.float32),
            pltpu.VMEM((_SIZE,), jnp.float32),
            pltpu.VMEM((_PICKS_PW,), jnp.int32),
            pltpu.VMEM((_PICKS_PW,), jnp.int32),
            pltpu.VMEM((_PICKS_PW,), jnp.float32),
            pltpu.VMEM((_PICKS_PW,), jnp.float32),
            pltpu.VMEM((_PICKS_PW,), jnp.int32),
            pltpu.VMEM((_RPW,), jnp.int32),
            pltpu.VMEM((16,), jnp.float32),
            pltpu.SemaphoreType.DMA,
            pltpu.SemaphoreType.DMA,
            pltpu.SemaphoreType.DMA,
        ],
    )
    return f(x, xflat, t32)


@jax.jit
def kernel(x, target):
    t32 = target.astype(jnp.int32)
    sc_part = _sc_call(x, x.reshape(-1), t32)
    tc_part = _tc_call(t32, x)
    return tc_part[0, 0] + jnp.sum(sc_part)


# speedup vs baseline: 2.8424x; 2.8281x over previous
import math

import jax
import jax.numpy as jnp
from jax import lax
from jax.experimental import pallas as pl
from jax.experimental.pallas import tpu as pltpu
from jax.experimental.pallas import tpu_sc as plsc

_SIZE = 32000
_PAD = 0
_SMOOTH = 0.1
_CONF = 1.0 - _SMOOTH
_EPS = _SMOOTH / (_SIZE - 2)
_N = 4096
_C1 = _EPS * math.log(_EPS) * (_SIZE - 2) + _CONF * math.log(_CONF)

_NW = 32
_R_SC = 1024
_R_TC = _N - _R_SC
_RPW = _R_SC // _NW
_BM = 128
_BN = _SIZE


def _tc_kernel(t_ref, x_ref, o_ref):
    i = pl.program_id(0)

    @pl.when(i == 0)
    def _():
        o_ref[...] = jnp.zeros_like(o_ref)

    t = t_ref[...]
    x = x_ref[...]
    live = t != _PAD
    cols = jax.lax.broadcasted_iota(jnp.int32, (_BM, _BN), 1)
    y = jnp.where(cols == t, (_CONF / _EPS) * x, x)
    rowsum = jnp.sum(y, axis=1, keepdims=True)
    acc = jnp.sum(jnp.where(live, -_EPS, 0.0) * rowsum)
    extra = jnp.sum(jnp.where(live, 1.0, 0.0) * (_EPS * x[:, 0:1] + _C1))
    o_ref[...] += (acc + extra).reshape(1, 1)


def _tc_call(t32, x):
    return pl.pallas_call(
        _tc_kernel,
        grid=(_R_TC // _BM,),
        in_specs=[
            pl.BlockSpec((_BM, 1), lambda i: (i, 0)),
            pl.BlockSpec((_BM, _BN), lambda i: (i, 0)),
        ],
        out_specs=pl.BlockSpec((1, 1), lambda i: (0, 0)),
        out_shape=jax.ShapeDtypeStruct((1, 1), jnp.float32),
    )(t32.reshape(_N, 1), x)


def _sc_body(x_hbm, t_hbm, out_hbm, outw_hbm, rowbuf, rowbuf2, trows, outv,
             winbuf, sem0, sem1):
    wid = lax.axis_index("c") * 16 + lax.axis_index("s")
    acc = jnp.zeros((16,), jnp.float32)

    row_base = _R_TC + wid * _RPW
    pltpu.sync_copy(t_hbm.at[pl.ds(row_base, _RPW)], trows)

    def _row_sum(buf):
        def chunk_body(k, accs):
            a0, a1, a2, a3 = accs
            b = k * 256
            for u in range(4):
                a0 = a0 + buf[pl.ds(b + u * 64, 16)]
                a1 = a1 + buf[pl.ds(b + u * 64 + 16, 16)]
                a2 = a2 + buf[pl.ds(b + u * 64 + 32, 16)]
                a3 = a3 + buf[pl.ds(b + u * 64 + 48, 16)]
            return (a0, a1, a2, a3)

        z = jnp.zeros((16,), jnp.float32)
        a0, a1, a2, a3 = lax.fori_loop(0, _SIZE // 256, chunk_body,
                                       (z, z, z, z))
        return (a0 + a1) + (a2 + a3)

    bufs = (rowbuf, rowbuf2)
    sems = (sem0, sem1)
    pltpu.async_copy(x_hbm.at[row_base], rowbuf, sem0)

    def group_body(g, acc):
        tv = trows[pl.ds(g * 16, 16)]
        live = tv != _PAD
        wv = jnp.where(live, jnp.float32(-_EPS), jnp.float32(0.0))
        w0v = tv & -16
        r0 = row_base + g * 16
        for rr in range(16):
            cur, nxt = bufs[rr % 2], bufs[(rr + 1) % 2]
            scur, snxt = sems[rr % 2], sems[(rr + 1) % 2]
            nxt_row = jnp.minimum(r0 + rr + 1, _N - 1)
            pltpu.async_copy(x_hbm.at[nxt_row], nxt, snxt)
            pltpu.make_async_copy(x_hbm.at[0], cur, scur).wait()
            acc = acc + wv[rr] * _row_sum(cur)
            off = (g * 16 + rr) * 32
            winbuf[pl.ds(off, 16)] = cur[pl.ds(w0v[rr], 16)]
            winbuf[pl.ds(off + 16, 16)] = cur[pl.ds(0, 16)]
        return acc

    acc = lax.fori_loop(0, _RPW // 16, group_body, acc)
    pltpu.make_async_copy(x_hbm.at[0], rowbuf, sem0).wait()

    outv[...] = acc
    pltpu.sync_copy(outv, out_hbm.at[wid])
    pltpu.sync_copy(winbuf, outw_hbm.at[wid])


def _sc_call(x, t32):
    mesh = plsc.VectorSubcoreMesh(core_axis_name="c", subcore_axis_name="s")
    f = pl.kernel(
        _sc_body,
        mesh=mesh,
        out_type=[
            jax.ShapeDtypeStruct((_NW, 16), jnp.float32),
            jax.ShapeDtypeStruct((_NW, _RPW * 32), jnp.float32),
        ],
        scratch_types=[
            pltpu.VMEM((_SIZE,), jnp.float32),
            pltpu.VMEM((_SIZE,), jnp.float32),
            pltpu.VMEM((_RPW,), jnp.int32),
            pltpu.VMEM((16,), jnp.float32),
            pltpu.VMEM((_RPW * 32,), jnp.float32),
            pltpu.SemaphoreType.DMA,
            pltpu.SemaphoreType.DMA,
        ],
    )
    return f(x, t32)


_FR = _R_SC * 32 // 128


def _fix_kernel(w_ref, v_ref, o_ref):
    w = w_ref[...]
    v = v_ref[...]
    cols = jax.lax.broadcasted_iota(jnp.int32, (_FR, 128), 1)
    c1s = jnp.where(cols % 32 == 16, w, 0.0)
    o_ref[...] = (jnp.sum(w * v)
                  + (_C1 / _EPS) * jnp.sum(c1s)).reshape(1, 1)


def _fix_call(wcol, wins):
    return pl.pallas_call(
        _fix_kernel,
        out_shape=jax.ShapeDtypeStruct((1, 1), jnp.float32),
    )(wcol.reshape(_FR, 128), wins.reshape(_FR, 128))


@jax.jit
def kernel(x, target):
    t32 = target.astype(jnp.int32)
    sc_sums, sc_wins = _sc_call(x, t32)
    tc_part = _tc_call(t32, x)
    tsc = t32[_R_TC:]
    live = (tsc != _PAD)[:, None]
    slots = jax.lax.broadcasted_iota(jnp.int32, (_R_SC, 32), 1)
    wcol = jnp.where(
        live & (slots == (tsc & 15)[:, None]), _EPS - _CONF,
        jnp.where(live & (slots == 16), _EPS, 0.0)).astype(jnp.float32)
    fix_part = _fix_call(wcol, sc_wins)
    return tc_part[0, 0] + jnp.sum(sc_sums) + fix_part[0, 0]


# speedup vs baseline: 2.8711x; 1.0101x over previous
import math

import jax
import jax.numpy as jnp
from jax import lax
from jax.experimental import pallas as pl
from jax.experimental.pallas import tpu as pltpu
from jax.experimental.pallas import tpu_sc as plsc

_SIZE = 32000
_PAD = 0
_SMOOTH = 0.1
_CONF = 1.0 - _SMOOTH
_EPS = _SMOOTH / (_SIZE - 2)
_N = 4096
_C1 = _EPS * math.log(_EPS) * (_SIZE - 2) + _CONF * math.log(_CONF)

_NW = 32
_R_SC = 512
_R_TC = _N - _R_SC
_RPW = _R_SC // _NW
_BM = 128
_BN = _SIZE


def _tc_kernel(t_ref, x_ref, o_ref):
    i = pl.program_id(0)

    @pl.when(i == 0)
    def _():
        o_ref[...] = jnp.zeros_like(o_ref)

    t = t_ref[...]
    x = x_ref[...]
    live = t != _PAD
    cols = jax.lax.broadcasted_iota(jnp.int32, (_BM, _BN), 1)
    y = jnp.where(cols == t, (_CONF / _EPS) * x, x)
    rowsum = jnp.sum(y, axis=1, keepdims=True)
    acc = jnp.sum(jnp.where(live, -_EPS, 0.0) * rowsum)
    extra = jnp.sum(jnp.where(live, 1.0, 0.0) * (_EPS * x[:, 0:1] + _C1))
    o_ref[...] += (acc + extra).reshape(1, 1)


def _tc_call(t32, x):
    return pl.pallas_call(
        _tc_kernel,
        grid=(_R_TC // _BM,),
        in_specs=[
            pl.BlockSpec((_BM, 1), lambda i: (i, 0)),
            pl.BlockSpec((_BM, _BN), lambda i: (i, 0)),
        ],
        out_specs=pl.BlockSpec((1, 1), lambda i: (0, 0)),
        out_shape=jax.ShapeDtypeStruct((1, 1), jnp.float32),
    )(t32.reshape(_N, 1), x)


def _sc_body(x_hbm, t_hbm, out_hbm, outw_hbm, rowbuf, rowbuf2, trows, outv,
             winbuf, sem0, sem1):
    wid = lax.axis_index("c") * 16 + lax.axis_index("s")
    acc = jnp.zeros((16,), jnp.float32)

    row_base = _R_TC + wid * _RPW
    pltpu.sync_copy(t_hbm.at[pl.ds(row_base, _RPW)], trows)

    def _row_sum(buf):
        def chunk_body(k, accs):
            a0, a1, a2, a3 = accs
            b = k * 256
            for u in range(4):
                a0 = a0 + buf[pl.ds(b + u * 64, 16)]
                a1 = a1 + buf[pl.ds(b + u * 64 + 16, 16)]
                a2 = a2 + buf[pl.ds(b + u * 64 + 32, 16)]
                a3 = a3 + buf[pl.ds(b + u * 64 + 48, 16)]
            return (a0, a1, a2, a3)

        z = jnp.zeros((16,), jnp.float32)
        a0, a1, a2, a3 = lax.fori_loop(0, _SIZE // 256, chunk_body,
                                       (z, z, z, z))
        return (a0 + a1) + (a2 + a3)

    bufs = (rowbuf, rowbuf2)
    sems = (sem0, sem1)
    pltpu.async_copy(x_hbm.at[row_base], rowbuf, sem0)

    def group_body(g, acc):
        tv = trows[pl.ds(g * 16, 16)]
        live = tv != _PAD
        wv = jnp.where(live, jnp.float32(-_EPS), jnp.float32(0.0))
        w0v = tv & -16
        r0 = row_base + g * 16
        for rr in range(16):
            cur, nxt = bufs[rr % 2], bufs[(rr + 1) % 2]
            scur, snxt = sems[rr % 2], sems[(rr + 1) % 2]
            nxt_row = jnp.minimum(r0 + rr + 1, _N - 1)
            pltpu.async_copy(x_hbm.at[nxt_row], nxt, snxt)
            pltpu.make_async_copy(x_hbm.at[0], cur, scur).wait()
            acc = acc + wv[rr] * _row_sum(cur)
            off = (g * 16 + rr) * 32
            winbuf[pl.ds(off, 16)] = cur[pl.ds(w0v[rr], 16)]
            winbuf[pl.ds(off + 16, 16)] = cur[pl.ds(0, 16)]
        return acc

    acc = lax.fori_loop(0, _RPW // 16, group_body, acc)
    pltpu.make_async_copy(x_hbm.at[0], rowbuf, sem0).wait()

    outv[...] = acc
    pltpu.sync_copy(outv, out_hbm.at[wid])
    pltpu.sync_copy(winbuf, outw_hbm.at[wid])


def _sc_call(x, t32):
    mesh = plsc.VectorSubcoreMesh(core_axis_name="c", subcore_axis_name="s")
    f = pl.kernel(
        _sc_body,
        mesh=mesh,
        out_type=[
            jax.ShapeDtypeStruct((_NW, 16), jnp.float32),
            jax.ShapeDtypeStruct((_NW, _RPW * 32), jnp.float32),
        ],
        scratch_types=[
            pltpu.VMEM((_SIZE,), jnp.float32),
            pltpu.VMEM((_SIZE,), jnp.float32),
            pltpu.VMEM((_RPW,), jnp.int32),
            pltpu.VMEM((16,), jnp.float32),
            pltpu.VMEM((_RPW * 32,), jnp.float32),
            pltpu.SemaphoreType.DMA,
            pltpu.SemaphoreType.DMA,
        ],
    )
    return f(x, t32)


_FR = _R_SC * 32 // 128


def _fix_kernel(w_ref, v_ref, o_ref):
    w = w_ref[...]
    v = v_ref[...]
    cols = jax.lax.broadcasted_iota(jnp.int32, (_FR, 128), 1)
    c1s = jnp.where(cols % 32 == 16, w, 0.0)
    o_ref[...] = (jnp.sum(w * v)
                  + (_C1 / _EPS) * jnp.sum(c1s)).reshape(1, 1)


def _fix_call(wcol, wins):
    return pl.pallas_call(
        _fix_kernel,
        out_shape=jax.ShapeDtypeStruct((1, 1), jnp.float32),
    )(wcol.reshape(_FR, 128), wins.reshape(_FR, 128))


@jax.jit
def kernel(x, target):
    t32 = target.astype(jnp.int32)
    sc_sums, sc_wins = _sc_call(x, t32)
    tc_part = _tc_call(t32, x)
    tsc = t32[_R_TC:]
    live = (tsc != _PAD)[:, None]
    slots = jax.lax.broadcasted_iota(jnp.int32, (_R_SC, 32), 1)
    wcol = jnp.where(
        live & (slots == (tsc & 15)[:, None]), _EPS - _CONF,
        jnp.where(live & (slots == 16), _EPS, 0.0)).astype(jnp.float32)
    fix_part = _fix_call(wcol, sc_wins)
    return tc_part[0, 0] + jnp.sum(sc_sums) + fix_part[0, 0]


# speedup vs baseline: 2.9083x; 1.0130x over previous
import math

import jax
import jax.numpy as jnp
from jax import lax
from jax.experimental import pallas as pl
from jax.experimental.pallas import tpu as pltpu
from jax.experimental.pallas import tpu_sc as plsc

_SIZE = 32000
_PAD = 0
_SMOOTH = 0.1
_CONF = 1.0 - _SMOOTH
_EPS = _SMOOTH / (_SIZE - 2)
_N = 4096
_C1 = _EPS * math.log(_EPS) * (_SIZE - 2) + _CONF * math.log(_CONF)

_NW = 16
_R_SC = 512
_R_TC = _N - _R_SC
_RPW = _R_SC // _NW
_BM = 128
_BN = _SIZE


def _tc_kernel(t_ref, x_ref, o_ref):
    i = pl.program_id(0)

    @pl.when(i == 0)
    def _():
        o_ref[...] = jnp.zeros_like(o_ref)

    t = t_ref[...]
    x = x_ref[...]
    live = t != _PAD
    cols = jax.lax.broadcasted_iota(jnp.int32, (_BM, _BN), 1)
    y = jnp.where(cols == t, (_CONF / _EPS) * x, x)
    rowsum = jnp.sum(y, axis=1, keepdims=True)
    acc = jnp.sum(jnp.where(live, -_EPS, 0.0) * rowsum)
    extra = jnp.sum(jnp.where(live, 1.0, 0.0) * (_EPS * x[:, 0:1] + _C1))
    o_ref[...] += (acc + extra).reshape(1, 1)


def _tc_call(t32, x):
    return pl.pallas_call(
        _tc_kernel,
        grid=(_R_TC // _BM,),
        in_specs=[
            pl.BlockSpec((_BM, 1), lambda i: (i, 0)),
            pl.BlockSpec((_BM, _BN), lambda i: (i, 0)),
        ],
        out_specs=pl.BlockSpec((1, 1), lambda i: (0, 0)),
        out_shape=jax.ShapeDtypeStruct((1, 1), jnp.float32),
    )(t32.reshape(_N, 1), x)


def _sc_body(x_hbm, t_hbm, out_hbm, outw_hbm, rowbuf, rowbuf2, trows, outv,
             winbuf, sem0, sem1):
    wid = lax.axis_index("c") * 16 + lax.axis_index("s")
    acc = jnp.zeros((16,), jnp.float32)

    row_base = _R_TC + wid * _RPW
    pltpu.sync_copy(t_hbm.at[pl.ds(row_base, _RPW)], trows)

    def _row_sum(buf):
        def chunk_body(k, accs):
            a0, a1, a2, a3 = accs
            b = k * 256
            for u in range(4):
                a0 = a0 + buf[pl.ds(b + u * 64, 16)]
                a1 = a1 + buf[pl.ds(b + u * 64 + 16, 16)]
                a2 = a2 + buf[pl.ds(b + u * 64 + 32, 16)]
                a3 = a3 + buf[pl.ds(b + u * 64 + 48, 16)]
            return (a0, a1, a2, a3)

        z = jnp.zeros((16,), jnp.float32)
        a0, a1, a2, a3 = lax.fori_loop(0, _SIZE // 256, chunk_body,
                                       (z, z, z, z))
        return (a0 + a1) + (a2 + a3)

    bufs = (rowbuf, rowbuf2)
    sems = (sem0, sem1)
    pltpu.async_copy(x_hbm.at[row_base], rowbuf, sem0)

    def group_body(g, acc):
        tv = trows[pl.ds(g * 16, 16)]
        live = tv != _PAD
        wv = jnp.where(live, jnp.float32(-_EPS), jnp.float32(0.0))
        w0v = tv & -16
        r0 = row_base + g * 16
        for rr in range(16):
            cur, nxt = bufs[rr % 2], bufs[(rr + 1) % 2]
            scur, snxt = sems[rr % 2], sems[(rr + 1) % 2]
            nxt_row = jnp.minimum(r0 + rr + 1, _N - 1)
            pltpu.async_copy(x_hbm.at[nxt_row], nxt, snxt)
            pltpu.make_async_copy(x_hbm.at[0], cur, scur).wait()
            acc = acc + wv[rr] * _row_sum(cur)
            off = (g * 16 + rr) * 32
            winbuf[pl.ds(off, 16)] = cur[pl.ds(w0v[rr], 16)]
            winbuf[pl.ds(off + 16, 16)] = cur[pl.ds(0, 16)]
        return acc

    acc = lax.fori_loop(0, _RPW // 16, group_body, acc)
    pltpu.make_async_copy(x_hbm.at[0], rowbuf, sem0).wait()

    outv[...] = acc
    pltpu.sync_copy(outv, out_hbm.at[wid])
    pltpu.sync_copy(winbuf, outw_hbm.at[wid])


def _sc_call(x, t32):
    mesh = plsc.VectorSubcoreMesh(core_axis_name="c", subcore_axis_name="s", num_cores=1)
    f = pl.kernel(
        _sc_body,
        mesh=mesh,
        out_type=[
            jax.ShapeDtypeStruct((_NW, 16), jnp.float32),
            jax.ShapeDtypeStruct((_NW, _RPW * 32), jnp.float32),
        ],
        scratch_types=[
            pltpu.VMEM((_SIZE,), jnp.float32),
            pltpu.VMEM((_SIZE,), jnp.float32),
            pltpu.VMEM((_RPW,), jnp.int32),
            pltpu.VMEM((16,), jnp.float32),
            pltpu.VMEM((_RPW * 32,), jnp.float32),
            pltpu.SemaphoreType.DMA,
            pltpu.SemaphoreType.DMA,
        ],
    )
    return f(x, t32)


_FR = _R_SC * 32 // 128


def _fix_kernel(w_ref, v_ref, o_ref):
    w = w_ref[...]
    v = v_ref[...]
    cols = jax.lax.broadcasted_iota(jnp.int32, (_FR, 128), 1)
    c1s = jnp.where(cols % 32 == 16, w, 0.0)
    o_ref[...] = (jnp.sum(w * v)
                  + (_C1 / _EPS) * jnp.sum(c1s)).reshape(1, 1)


def _fix_call(wcol, wins):
    return pl.pallas_call(
        _fix_kernel,
        out_shape=jax.ShapeDtypeStruct((1, 1), jnp.float32),
    )(wcol.reshape(_FR, 128), wins.reshape(_FR, 128))


@jax.jit
def kernel(x, target):
    t32 = target.astype(jnp.int32)
    sc_sums, sc_wins = _sc_call(x, t32)
    tc_part = _tc_call(t32, x)
    tsc = t32[_R_TC:]
    live = (tsc != _PAD)[:, None]
    slots = jax.lax.broadcasted_iota(jnp.int32, (_R_SC, 32), 1)
    wcol = jnp.where(
        live & (slots == (tsc & 15)[:, None]), _EPS - _CONF,
        jnp.where(live & (slots == 16), _EPS, 0.0)).astype(jnp.float32)
    fix_part = _fix_call(wcol, sc_wins)
    return tc_part[0, 0] + jnp.sum(sc_sums) + fix_part[0, 0]


# speedup vs baseline: 2.9296x; 1.0073x over previous
import math

import jax
import jax.numpy as jnp
from jax import lax
from jax.experimental import pallas as pl
from jax.experimental.pallas import tpu as pltpu
from jax.experimental.pallas import tpu_sc as plsc

_SIZE = 32000
_PAD = 0
_SMOOTH = 0.1
_CONF = 1.0 - _SMOOTH
_EPS = _SMOOTH / (_SIZE - 2)
_N = 4096
_C1 = _EPS * math.log(_EPS) * (_SIZE - 2) + _CONF * math.log(_CONF)

_NW = 16
_R_SC = 256
_R_TC = _N - _R_SC
_RPW = _R_SC // _NW
_BM = 128
_BN = _SIZE


def _tc_kernel(t_ref, x_ref, o_ref):
    i = pl.program_id(0)

    @pl.when(i == 0)
    def _():
        o_ref[...] = jnp.zeros_like(o_ref)

    t = t_ref[...]
    x = x_ref[...]
    live = t != _PAD
    cols = jax.lax.broadcasted_iota(jnp.int32, (_BM, _BN), 1)
    y = jnp.where(cols == t, (_CONF / _EPS) * x, x)
    rowsum = jnp.sum(y, axis=1, keepdims=True)
    acc = jnp.sum(jnp.where(live, -_EPS, 0.0) * rowsum)
    extra = jnp.sum(jnp.where(live, 1.0, 0.0) * (_EPS * x[:, 0:1] + _C1))
    o_ref[...] += (acc + extra).reshape(1, 1)


def _tc_call(t32, x):
    return pl.pallas_call(
        _tc_kernel,
        grid=(_R_TC // _BM,),
        in_specs=[
            pl.BlockSpec((_BM, 1), lambda i: (i, 0)),
            pl.BlockSpec((_BM, _BN), lambda i: (i, 0)),
        ],
        out_specs=pl.BlockSpec((1, 1), lambda i: (0, 0)),
        out_shape=jax.ShapeDtypeStruct((1, 1), jnp.float32),
    )(t32.reshape(_N, 1), x)


def _sc_body(x_hbm, t_hbm, out_hbm, outw_hbm, rowbuf, rowbuf2, trows, outv,
             winbuf, sem0, sem1):
    wid = lax.axis_index("c") * 16 + lax.axis_index("s")
    acc = jnp.zeros((16,), jnp.float32)

    row_base = _R_TC + wid * _RPW
    pltpu.sync_copy(t_hbm.at[pl.ds(row_base, _RPW)], trows)

    def _row_sum(buf):
        def chunk_body(k, accs):
            a0, a1, a2, a3 = accs
            b = k * 256
            for u in range(4):
                a0 = a0 + buf[pl.ds(b + u * 64, 16)]
                a1 = a1 + buf[pl.ds(b + u * 64 + 16, 16)]
                a2 = a2 + buf[pl.ds(b + u * 64 + 32, 16)]
                a3 = a3 + buf[pl.ds(b + u * 64 + 48, 16)]
            return (a0, a1, a2, a3)

        z = jnp.zeros((16,), jnp.float32)
        a0, a1, a2, a3 = lax.fori_loop(0, _SIZE // 256, chunk_body,
                                       (z, z, z, z))
        return (a0 + a1) + (a2 + a3)

    bufs = (rowbuf, rowbuf2)
    sems = (sem0, sem1)
    pltpu.async_copy(x_hbm.at[row_base], rowbuf, sem0)

    def group_body(g, acc):
        tv = trows[pl.ds(g * 16, 16)]
        live = tv != _PAD
        wv = jnp.where(live, jnp.float32(-_EPS), jnp.float32(0.0))
        w0v = tv & -16
        r0 = row_base + g * 16
        for rr in range(16):
            cur, nxt = bufs[rr % 2], bufs[(rr + 1) % 2]
            scur, snxt = sems[rr % 2], sems[(rr + 1) % 2]
            nxt_row = jnp.minimum(r0 + rr + 1, _N - 1)
            pltpu.async_copy(x_hbm.at[nxt_row], nxt, snxt)
            pltpu.make_async_copy(x_hbm.at[0], cur, scur).wait()
            acc = acc + wv[rr] * _row_sum(cur)
            off = (g * 16 + rr) * 32
            winbuf[pl.ds(off, 16)] = cur[pl.ds(w0v[rr], 16)]
            winbuf[pl.ds(off + 16, 16)] = cur[pl.ds(0, 16)]
        return acc

    acc = lax.fori_loop(0, _RPW // 16, group_body, acc)
    pltpu.make_async_copy(x_hbm.at[0], rowbuf, sem0).wait()

    outv[...] = acc
    pltpu.sync_copy(outv, out_hbm.at[wid])
    pltpu.sync_copy(winbuf, outw_hbm.at[wid])


def _sc_call(x, t32):
    mesh = plsc.VectorSubcoreMesh(core_axis_name="c", subcore_axis_name="s", num_cores=1)
    f = pl.kernel(
        _sc_body,
        mesh=mesh,
        out_type=[
            jax.ShapeDtypeStruct((_NW, 16), jnp.float32),
            jax.ShapeDtypeStruct((_NW, _RPW * 32), jnp.float32),
        ],
        scratch_types=[
            pltpu.VMEM((_SIZE,), jnp.float32),
            pltpu.VMEM((_SIZE,), jnp.float32),
            pltpu.VMEM((_RPW,), jnp.int32),
            pltpu.VMEM((16,), jnp.float32),
            pltpu.VMEM((_RPW * 32,), jnp.float32),
            pltpu.SemaphoreType.DMA,
            pltpu.SemaphoreType.DMA,
        ],
    )
    return f(x, t32)


_FR = _R_SC * 32 // 128


def _fix_kernel(w_ref, v_ref, o_ref):
    w = w_ref[...]
    v = v_ref[...]
    cols = jax.lax.broadcasted_iota(jnp.int32, (_FR, 128), 1)
    c1s = jnp.where(cols % 32 == 16, w, 0.0)
    o_ref[...] = (jnp.sum(w * v)
                  + (_C1 / _EPS) * jnp.sum(c1s)).reshape(1, 1)


def _fix_call(wcol, wins):
    return pl.pallas_call(
        _fix_kernel,
        out_shape=jax.ShapeDtypeStruct((1, 1), jnp.float32),
    )(wcol.reshape(_FR, 128), wins.reshape(_FR, 128))


@jax.jit
def kernel(x, target):
    t32 = target.astype(jnp.int32)
    sc_sums, sc_wins = _sc_call(x, t32)
    tc_part = _tc_call(t32, x)
    tsc = t32[_R_TC:]
    live = (tsc != _PAD)[:, None]
    slots = jax.lax.broadcasted_iota(jnp.int32, (_R_SC, 32), 1)
    wcol = jnp.where(
        live & (slots == (tsc & 15)[:, None]), _EPS - _CONF,
        jnp.where(live & (slots == 16), _EPS, 0.0)).astype(jnp.float32)
    fix_part = _fix_call(wcol, sc_wins)
    return tc_part[0, 0] + jnp.sum(sc_sums) + fix_part[0, 0]
